# Initial kernel scaffold; baseline (speedup 1.0000x reference)
import jax
import jax.numpy as jnp
from jax.experimental import pallas as pl


def kernel(x):
    raise NotImplementedError("write your pallas kernel here")


# speedup vs baseline: 2.7552x; 2.7552x over previous
import jax
import jax.numpy as jnp
from jax.experimental import pallas as pl
from jax.experimental.pallas import tpu as pltpu

_PERIOD = 4
_START = 2
_ROWS = 128
_L = 128


def _decimate_block(x_ref, o_ref):
    r, t = x_ref.shape
    xb = x_ref[...]
    xg = pltpu.einshape("a(bc)->bac", xb, c=_L)
    idx = (_PERIOD * jax.lax.broadcasted_iota(jnp.int32, xg.shape, 2) + _START) % _L
    g = jnp.take_along_axis(xg, idx, axis=2)
    gh = pltpu.einshape("(hi)ac->hiac", g, i=_PERIOD)
    j = jax.lax.broadcasted_iota(jnp.int32, (t // (_PERIOD * _L), r, _L), 2)
    q = _L // _PERIOD
    out = jnp.where(
        j < q,
        gh[:, 0],
        jnp.where(
            j < 2 * q,
            gh[:, 1],
            jnp.where(j < 3 * q, gh[:, 2], gh[:, 3]),
        ),
    )
    o_ref[...] = pltpu.einshape("bac->a(bc)", out)


def kernel(x):
    b, n, t = x.shape
    k = (t - _START + _PERIOD - 1) // _PERIOD
    xf = x.reshape(b * n, t)
    yf = pl.pallas_call(
        _decimate_block,
        grid=(b * n // _ROWS,),
        in_specs=[pl.BlockSpec((_ROWS, t), lambda i: (i, 0))],
        out_specs=pl.BlockSpec((_ROWS, k), lambda i: (i, 0)),
        out_shape=jax.ShapeDtypeStruct((b * n, k), x.dtype),
    )(xf)
    return yf.reshape(b, n, k)


# speedup vs baseline: 3.1551x; 1.1452x over previous
import jax
import jax.numpy as jnp
from jax.experimental import pallas as pl
from jax.experimental.pallas import tpu as pltpu

_PERIOD = 4
_START = 2
_ROWS = 256
_L = 128


def _decimate_block(x_ref, o_ref):
    r, t = x_ref.shape
    xb = x_ref[...]
    xg = pltpu.einshape("a(bc)->bac", xb, c=_L)
    idx = (_PERIOD * jax.lax.broadcasted_iota(jnp.int32, xg.shape, 2) + _START) % _L
    g = jnp.take_along_axis(xg, idx, axis=2)
    gh = pltpu.einshape("(hi)ac->hiac", g, i=_PERIOD)
    j = jax.lax.broadcasted_iota(jnp.int32, (t // (_PERIOD * _L), r, _L), 2)
    q = _L // _PERIOD
    out = jnp.where(
        j < q,
        gh[:, 0],
        jnp.where(
            j < 2 * q,
            gh[:, 1],
            jnp.where(j < 3 * q, gh[:, 2], gh[:, 3]),
        ),
    )
    o_ref[...] = pltpu.einshape("bac->a(bc)", out)


def kernel(x):
    b, n, t = x.shape
    k = (t - _START + _PERIOD - 1) // _PERIOD
    xf = x.reshape(b * n, t)
    yf = pl.pallas_call(
        _decimate_block,
        grid=(b * n // _ROWS,),
        in_specs=[pl.BlockSpec((_ROWS, t), lambda i: (i, 0))],
        out_specs=pl.BlockSpec((_ROWS, k), lambda i: (i, 0)),
        out_shape=jax.ShapeDtypeStruct((b * n, k), x.dtype),
    )(xf)
    return yf.reshape(b, n, k)


# speedup vs baseline: 3.2668x; 1.0354x over previous
import jax
import jax.numpy as jnp
from jax.experimental import pallas as pl
from jax.experimental.pallas import tpu as pltpu

_PERIOD = 4
_START = 2
_ROWS = 512
_L = 128


def _decimate_block(x_ref, o_ref):
    r, t = x_ref.shape
    xb = x_ref[...]
    xg = pltpu.einshape("a(bc)->bac", xb, c=_L)
    idx = (_PERIOD * jax.lax.broadcasted_iota(jnp.int32, xg.shape, 2) + _START) % _L
    g = jnp.take_along_axis(xg, idx, axis=2)
    gh = pltpu.einshape("(hi)ac->hiac", g, i=_PERIOD)
    j = jax.lax.broadcasted_iota(jnp.int32, (t // (_PERIOD * _L), r, _L), 2)
    q = _L // _PERIOD
    out = jnp.where(
        j < q,
        gh[:, 0],
        jnp.where(
            j < 2 * q,
            gh[:, 1],
            jnp.where(j < 3 * q, gh[:, 2], gh[:, 3]),
        ),
    )
    o_ref[...] = pltpu.einshape("bac->a(bc)", out)


def kernel(x):
    b, n, t = x.shape
    k = (t - _START + _PERIOD - 1) // _PERIOD
    xf = x.reshape(b * n, t)
    yf = pl.pallas_call(
        _decimate_block,
        grid=(b * n // _ROWS,),
        in_specs=[pl.BlockSpec((_ROWS, t), lambda i: (i, 0))],
        out_specs=pl.BlockSpec((_ROWS, k), lambda i: (i, 0)),
        out_shape=jax.ShapeDtypeStruct((b * n, k), x.dtype),
    )(xf)
    return yf.reshape(b, n, k)
